# baseline (speedup 1.0000x reference)
import jax
import jax.numpy as jnp
from jax.experimental import pallas as pl
from jax.experimental.pallas import tpu as pltpu

_T = 16
_BLK = 256


def _minmax_body(x_ref, min_ref, max_ref):
    i = pl.program_id(0)
    bmin = jnp.min(x_ref[...])
    bmax = jnp.max(x_ref[...])

    @pl.when(i == 0)
    def _init():
        min_ref[0, 0] = bmin
        max_ref[0, 0] = bmax

    @pl.when(i > 0)
    def _acc():
        min_ref[0, 0] = jnp.minimum(min_ref[0, 0], bmin)
        max_ref[0, 0] = jnp.maximum(max_ref[0, 0], bmax)


def _encode_body(min_ref, max_ref, x_ref, out_ref):
    mn = min_ref[0, 0]
    mx = max_ref[0, 0]
    x = x_ref[...]
    xn = jnp.clip((x - mn) / (mx - mn + 1e-8), 0.0, 1.0)
    lat = ((1.0 - xn) * (_T - 1)).astype(jnp.int32)
    t = jax.lax.broadcasted_iota(jnp.int32, (x.shape[0], _T, x.shape[1]), 1)
    out_ref[...] = (lat[:, None, :] == t).astype(jnp.float32)


def kernel(x):
    B, F = x.shape
    mn, mx = pl.pallas_call(
        _minmax_body,
        grid=(B // 1024,),
        in_specs=(pl.BlockSpec((1024, F), lambda i: (i, 0)),),
        out_specs=(
            pl.BlockSpec(memory_space=pltpu.SMEM, block_shape=(1, 1), index_map=lambda i: (0, 0)),
            pl.BlockSpec(memory_space=pltpu.SMEM, block_shape=(1, 1), index_map=lambda i: (0, 0)),
        ),
        out_shape=(
            jax.ShapeDtypeStruct((1, 1), jnp.float32),
            jax.ShapeDtypeStruct((1, 1), jnp.float32),
        ),
    )(x)

    spikes = pl.pallas_call(
        _encode_body,
        grid=(B // _BLK,),
        in_specs=(
            pl.BlockSpec(memory_space=pltpu.SMEM),
            pl.BlockSpec(memory_space=pltpu.SMEM),
            pl.BlockSpec((_BLK, F), lambda i: (i, 0)),
        ),
        out_specs=pl.BlockSpec((_BLK, _T, F), lambda i: (i, 0, 0)),
        out_shape=jax.ShapeDtypeStruct((B, _T, F), jnp.float32),
    )(mn, mx, x)
    return spikes


# speedup vs baseline: 1.0107x; 1.0107x over previous
import jax
import jax.numpy as jnp
from jax.experimental import pallas as pl
from jax.experimental.pallas import tpu as pltpu

_T = 16
_BLK = 256


def _minmax_body(x_ref, min_ref, max_ref):
    i = pl.program_id(0)
    bmin = jnp.min(x_ref[...])
    bmax = jnp.max(x_ref[...])

    @pl.when(i == 0)
    def _init():
        min_ref[0, 0] = bmin
        max_ref[0, 0] = bmax

    @pl.when(i > 0)
    def _acc():
        min_ref[0, 0] = jnp.minimum(min_ref[0, 0], bmin)
        max_ref[0, 0] = jnp.maximum(max_ref[0, 0], bmax)


def _encode_body(min_ref, max_ref, x_ref, out_ref):
    mn = min_ref[0, 0]
    mx = max_ref[0, 0]
    x = x_ref[...]
    xn = jnp.clip((x - mn) / (mx - mn + 1e-8), 0.0, 1.0)
    lat = ((1.0 - xn) * (_T - 1)).astype(jnp.int32)
    t = jax.lax.broadcasted_iota(jnp.int32, (x.shape[0], _T, x.shape[1]), 1)
    out_ref[...] = (lat[:, None, :] == t).astype(jnp.float32)


def kernel(x):
    B, F = x.shape
    mn, mx = pl.pallas_call(
        _minmax_body,
        grid=(B // 2048,),
        in_specs=(pl.BlockSpec((2048, F), lambda i: (i, 0)),),
        out_specs=(
            pl.BlockSpec(memory_space=pltpu.SMEM, block_shape=(1, 1), index_map=lambda i: (0, 0)),
            pl.BlockSpec(memory_space=pltpu.SMEM, block_shape=(1, 1), index_map=lambda i: (0, 0)),
        ),
        out_shape=(
            jax.ShapeDtypeStruct((1, 1), jnp.float32),
            jax.ShapeDtypeStruct((1, 1), jnp.float32),
        ),
    )(x)

    spikes = pl.pallas_call(
        _encode_body,
        grid=(B // _BLK,),
        in_specs=(
            pl.BlockSpec(memory_space=pltpu.SMEM),
            pl.BlockSpec(memory_space=pltpu.SMEM),
            pl.BlockSpec((_BLK, F), lambda i: (i, 0)),
        ),
        out_specs=pl.BlockSpec((_BLK, _T, F), lambda i: (i, 0, 0)),
        out_shape=jax.ShapeDtypeStruct((B, _T, F), jnp.float32),
    )(mn, mx, x)
    return spikes
